# baseline (speedup 1.0000x reference)
import functools

import jax
import jax.numpy as jnp
from jax import lax
from jax.experimental import pallas as pl
from jax.experimental.pallas import tpu as pltpu

B, D_IN, C, NC = 16384, 500, 64, 2
BLK = 4096
NB = B // BLK
NSPLIT = 4
SUB = BLK // NSPLIT

_CONTRACT_D0 = (((0,), (0,)), ((), ()))


def _leaky(x):
    return jnp.where(x >= 0, x, 0.01 * x)


def _fused_kernel(xt_hbm, labels_ref, w1_ref, b1_ref, w2t_ref, b2_ref,
                  w3_ref, b3_ref, w4t_ref, b4_ref,
                  logits_t_ref, preds_t_ref, proj_t_ref, vec_t_ref, loss_ref,
                  x_buf, dma_sems):
    b = pl.program_id(0)

    def copies(slot, blk):
        return [
            pltpu.make_async_copy(
                xt_hbm.at[:, pl.ds(blk * BLK + j * SUB, SUB)],
                x_buf.at[slot, :, pl.ds(j * SUB, SUB)],
                dma_sems.at[slot, j],
            )
            for j in range(NSPLIT)
        ]

    @pl.when(b == 0)
    def _():
        for c in copies(0, 0):
            c.start()

    @pl.when(b + 1 < NB)
    def _():
        nxt = (b + 1) % 2
        for c in copies(nxt, b + 1):
            c.start()

    for c in copies(b % 2, b):
        c.wait()

    xt = x_buf[b % 2]
    h_t = lax.dot_general(w1_ref[...], xt, _CONTRACT_D0,
                          preferred_element_type=jnp.float32)
    h_t = _leaky(h_t + b1_ref[...].T)
    proj_t = jnp.dot(w2t_ref[...], h_t,
                     preferred_element_type=jnp.float32)
    proj_t = proj_t + b2_ref[...].T
    proj_t_ref[...] = proj_t
    vec_t_ref[...] = proj_t

    c_t = lax.dot_general(w3_ref[...], proj_t, _CONTRACT_D0,
                          preferred_element_type=jnp.float32)
    c_t = _leaky(c_t + b3_ref[...].T)
    lr_t = jnp.dot(w4t_ref[...], c_t,
                   preferred_element_type=jnp.float32)
    lr_t = lr_t + b4_ref[...].T

    lr0 = lr_t[0:1, :]
    lr1 = lr_t[1:2, :]
    m = jnp.maximum(lr0, lr1)
    e0 = jnp.exp(lr0 - m)
    e1 = jnp.exp(lr1 - m)
    s = e0 + e1
    inv_s = 1.0 / s
    logits_t_ref[...] = jnp.concatenate([e0 * inv_s, e1 * inv_s], axis=0)

    preds_t_ref[...] = (lr1 > lr0).astype(jnp.int32)

    log_s = jnp.log(s)
    logp0 = lr0 - m - log_s
    logp1 = lr1 - m - log_s
    lab = labels_ref[...]
    picked = jnp.where(lab == 0, logp0, logp1)
    partial = (jnp.sum(picked) * (-1.0 / B)).reshape(1, 1)

    @pl.when(b == 0)
    def _():
        loss_ref[...] = jnp.zeros((1, 1), jnp.float32)

    loss_ref[...] += partial


@functools.partial(jax.jit, static_argnames=())
def kernel(texts, img, labels, W1, b1, W2, b2, W3, b3, W4, b4):
    del img
    grid = (NB,)
    texts_t = texts.T
    labels2 = labels.reshape(1, B).astype(jnp.int32)

    out_shapes = (
        jax.ShapeDtypeStruct((NC, B), jnp.float32),
        jax.ShapeDtypeStruct((1, B), jnp.int32),
        jax.ShapeDtypeStruct((C, B), jnp.float32),
        jax.ShapeDtypeStruct((C, B), jnp.float32),
        jax.ShapeDtypeStruct((1, 1), jnp.float32),
    )

    full = lambda *dims: pl.BlockSpec(dims, lambda i: (0,) * len(dims))
    in_specs = [
        pl.BlockSpec(memory_space=pl.ANY),
        pl.BlockSpec((1, BLK), lambda i: (0, i)),
        full(D_IN, 2 * C),
        full(1, 2 * C),
        full(C, 2 * C),
        full(1, C),
        full(C, C),
        full(1, C),
        full(NC, C),
        full(1, NC),
    ]
    out_specs = (
        pl.BlockSpec((NC, BLK), lambda i: (0, i)),
        pl.BlockSpec((1, BLK), lambda i: (0, i)),
        pl.BlockSpec((C, BLK), lambda i: (0, i)),
        pl.BlockSpec((C, BLK), lambda i: (0, i)),
        pl.BlockSpec((1, 1), lambda i: (0, 0)),
    )

    logits_t, preds_t, proj_t, vec_t, loss_sum = pl.pallas_call(
        _fused_kernel,
        grid=grid,
        in_specs=in_specs,
        out_specs=out_specs,
        out_shape=out_shapes,
        scratch_shapes=[
            pltpu.VMEM((2, D_IN, BLK), jnp.float32),
            pltpu.SemaphoreType.DMA((2, NSPLIT)),
        ],
    )(texts_t, labels2, W1, b1.reshape(1, -1), W2.T, b2.reshape(1, -1),
      W3, b3.reshape(1, -1), W4.T, b4.reshape(1, -1))

    logits = logits_t.T
    preds = preds_t.reshape(B)
    projections = proj_t.T
    vectors = vec_t.T[:, None, :]
    loss = loss_sum.reshape(())
    return (logits, preds, projections, vectors, loss)


# speedup vs baseline: 1.0073x; 1.0073x over previous
import functools

import jax
import jax.numpy as jnp
from jax import lax
from jax.experimental import pallas as pl
from jax.experimental.pallas import tpu as pltpu

B, D_IN, C, NC = 16384, 500, 64, 2
BLK = 4096
NB = B // BLK
_ROW_SPLITS = ((0, 128), (128, 128), (256, 128), (384, 116))
NSPLIT = len(_ROW_SPLITS)

_CONTRACT_D0 = (((0,), (0,)), ((), ()))


def _leaky(x):
    return jnp.where(x >= 0, x, 0.01 * x)


def _fused_kernel(xt_hbm, labels_ref, w1_ref, b1_ref, w2t_ref, b2_ref,
                  w3_ref, b3_ref, w4t_ref, b4_ref,
                  logits_t_ref, preds_t_ref, proj_t_ref, vec_t_ref, loss_ref,
                  x_buf, dma_sems):
    b = pl.program_id(0)

    def copies(slot, blk):
        return [
            pltpu.make_async_copy(
                xt_hbm.at[pl.ds(r0, nr), pl.ds(blk * BLK, BLK)],
                x_buf.at[slot, pl.ds(r0, nr), :],
                dma_sems.at[slot, j],
            )
            for j, (r0, nr) in enumerate(_ROW_SPLITS)
        ]

    @pl.when(b == 0)
    def _():
        for c in copies(0, 0):
            c.start()

    @pl.when(b + 1 < NB)
    def _():
        nxt = (b + 1) % 2
        for c in copies(nxt, b + 1):
            c.start()

    for c in copies(b % 2, b):
        c.wait()

    xt = x_buf[b % 2]
    h_t = lax.dot_general(w1_ref[...], xt, _CONTRACT_D0,
                          preferred_element_type=jnp.float32)
    h_t = _leaky(h_t + b1_ref[...].T)
    proj_t = jnp.dot(w2t_ref[...], h_t,
                     preferred_element_type=jnp.float32)
    proj_t = proj_t + b2_ref[...].T
    proj_t_ref[...] = proj_t
    vec_t_ref[...] = proj_t

    c_t = lax.dot_general(w3_ref[...], proj_t, _CONTRACT_D0,
                          preferred_element_type=jnp.float32)
    c_t = _leaky(c_t + b3_ref[...].T)
    lr_t = jnp.dot(w4t_ref[...], c_t,
                   preferred_element_type=jnp.float32)
    lr_t = lr_t + b4_ref[...].T

    lr0 = lr_t[0:1, :]
    lr1 = lr_t[1:2, :]
    m = jnp.maximum(lr0, lr1)
    e0 = jnp.exp(lr0 - m)
    e1 = jnp.exp(lr1 - m)
    s = e0 + e1
    inv_s = 1.0 / s
    logits_t_ref[...] = jnp.concatenate([e0 * inv_s, e1 * inv_s], axis=0)

    preds_t_ref[...] = (lr1 > lr0).astype(jnp.int32)

    log_s = jnp.log(s)
    logp0 = lr0 - m - log_s
    logp1 = lr1 - m - log_s
    lab = labels_ref[...]
    picked = jnp.where(lab == 0, logp0, logp1)
    partial = (jnp.sum(picked) * (-1.0 / B)).reshape(1, 1)

    @pl.when(b == 0)
    def _():
        loss_ref[...] = jnp.zeros((1, 1), jnp.float32)

    loss_ref[...] += partial


@functools.partial(jax.jit, static_argnames=())
def kernel(texts, img, labels, W1, b1, W2, b2, W3, b3, W4, b4):
    del img
    grid = (NB,)
    texts_t = texts.T
    labels2 = labels.reshape(1, B).astype(jnp.int32)

    out_shapes = (
        jax.ShapeDtypeStruct((NC, B), jnp.float32),
        jax.ShapeDtypeStruct((1, B), jnp.int32),
        jax.ShapeDtypeStruct((C, B), jnp.float32),
        jax.ShapeDtypeStruct((C, B), jnp.float32),
        jax.ShapeDtypeStruct((1, 1), jnp.float32),
    )

    full = lambda *dims: pl.BlockSpec(dims, lambda i: (0,) * len(dims))
    in_specs = [
        pl.BlockSpec(memory_space=pl.ANY),
        pl.BlockSpec((1, BLK), lambda i: (0, i)),
        full(D_IN, 2 * C),
        full(1, 2 * C),
        full(C, 2 * C),
        full(1, C),
        full(C, C),
        full(1, C),
        full(NC, C),
        full(1, NC),
    ]
    out_specs = (
        pl.BlockSpec((NC, BLK), lambda i: (0, i)),
        pl.BlockSpec((1, BLK), lambda i: (0, i)),
        pl.BlockSpec((C, BLK), lambda i: (0, i)),
        pl.BlockSpec((C, BLK), lambda i: (0, i)),
        pl.BlockSpec((1, 1), lambda i: (0, 0)),
    )

    logits_t, preds_t, proj_t, vec_t, loss_sum = pl.pallas_call(
        _fused_kernel,
        grid=grid,
        in_specs=in_specs,
        out_specs=out_specs,
        out_shape=out_shapes,
        scratch_shapes=[
            pltpu.VMEM((2, D_IN, BLK), jnp.float32),
            pltpu.SemaphoreType.DMA((2, NSPLIT)),
        ],
    )(texts_t, labels2, W1, b1.reshape(1, -1), W2.T, b2.reshape(1, -1),
      W3, b3.reshape(1, -1), W4.T, b4.reshape(1, -1))

    logits = logits_t.T
    preds = preds_t.reshape(B)
    projections = proj_t.T
    vectors = vec_t.T[:, None, :]
    loss = loss_sum.reshape(())
    return (logits, preds, projections, vectors, loss)


# speedup vs baseline: 1.0886x; 1.0807x over previous
import functools

import jax
import jax.numpy as jnp
from jax import lax
from jax.experimental import pallas as pl

B, D_IN, C, NC = 16384, 500, 64, 2
BLK = 4096

_CONTRACT_D0 = (((0,), (0,)), ((), ()))


def _leaky(x):
    return jnp.where(x >= 0, x, 0.01 * x)


def _fused_kernel(xt_ref, labels_ref, w1_ref, b1_ref, w2t_ref, b2_ref,
                  w3_ref, b3_ref, w4t_ref, b4_ref,
                  logits_t_ref, preds_t_ref, proj_t_ref, vec_t_ref, loss_ref):
    i = pl.program_id(0)

    xt = xt_ref[...]
    h_t = lax.dot_general(w1_ref[...], xt, _CONTRACT_D0,
                          preferred_element_type=jnp.float32)
    h_t = _leaky(h_t + b1_ref[...].T)
    proj_t = jnp.dot(w2t_ref[...], h_t,
                     preferred_element_type=jnp.float32)
    proj_t = proj_t + b2_ref[...].T
    proj_t_ref[...] = proj_t
    vec_t_ref[...] = proj_t

    c_t = lax.dot_general(w3_ref[...], proj_t, _CONTRACT_D0,
                          preferred_element_type=jnp.float32)
    c_t = _leaky(c_t + b3_ref[...].T)
    lr_t = jnp.dot(w4t_ref[...], c_t,
                   preferred_element_type=jnp.float32)
    lr_t = lr_t + b4_ref[...].T

    lr0 = lr_t[0:1, :]
    lr1 = lr_t[1:2, :]
    m = jnp.maximum(lr0, lr1)
    e0 = jnp.exp(lr0 - m)
    e1 = jnp.exp(lr1 - m)
    s = e0 + e1
    inv_s = 1.0 / s
    logits_t_ref[...] = jnp.concatenate([e0 * inv_s, e1 * inv_s], axis=0)

    preds_t_ref[...] = (lr1 > lr0).astype(jnp.int32)

    log_s = jnp.log(s)
    logp0 = lr0 - m - log_s
    logp1 = lr1 - m - log_s
    lab = labels_ref[...]
    picked = jnp.where(lab == 0, logp0, logp1)
    partial = (jnp.sum(picked) * (-1.0 / B)).reshape(1, 1)

    @pl.when(i == 0)
    def _():
        loss_ref[...] = jnp.zeros((1, 1), jnp.float32)

    loss_ref[...] += partial


@functools.partial(jax.jit, static_argnames=())
def kernel(texts, img, labels, W1, b1, W2, b2, W3, b3, W4, b4):
    del img
    grid = (B // BLK,)
    texts_t = texts.T
    labels2 = labels.reshape(1, B).astype(jnp.int32)

    out_shapes = (
        jax.ShapeDtypeStruct((NC, B), jnp.float32),
        jax.ShapeDtypeStruct((1, B), jnp.int32),
        jax.ShapeDtypeStruct((C, B), jnp.float32),
        jax.ShapeDtypeStruct((C, B), jnp.float32),
        jax.ShapeDtypeStruct((1, 1), jnp.float32),
    )

    full = lambda *dims: pl.BlockSpec(dims, lambda i: (0,) * len(dims))
    in_specs = [
        pl.BlockSpec((D_IN, BLK), lambda i: (0, i)),
        pl.BlockSpec((1, BLK), lambda i: (0, i)),
        full(D_IN, 2 * C),
        full(1, 2 * C),
        full(C, 2 * C),
        full(1, C),
        full(C, C),
        full(1, C),
        full(NC, C),
        full(1, NC),
    ]
    out_specs = (
        pl.BlockSpec((NC, BLK), lambda i: (0, i)),
        pl.BlockSpec((1, BLK), lambda i: (0, i)),
        pl.BlockSpec((C, BLK), lambda i: (0, i)),
        pl.BlockSpec((C, BLK), lambda i: (0, i)),
        pl.BlockSpec((1, 1), lambda i: (0, 0)),
    )

    logits_t, preds_t, proj_t, vec_t, loss_sum = pl.pallas_call(
        _fused_kernel,
        grid=grid,
        in_specs=in_specs,
        out_specs=out_specs,
        out_shape=out_shapes,
    )(texts_t, labels2, W1, b1.reshape(1, -1), W2.T, b2.reshape(1, -1),
      W3, b3.reshape(1, -1), W4.T, b4.reshape(1, -1))

    logits = logits_t.T
    preds = preds_t.reshape(B)
    projections = proj_t.T
    vectors = vec_t.T[:, None, :]
    loss = loss_sum.reshape(())
    return (logits, preds, projections, vectors, loss)
